# baseline (speedup 1.0000x reference)
import jax
import jax.numpy as jnp
from jax import lax
from jax.experimental import pallas as pl
from jax.experimental.pallas import tpu as pltpu
from jax.experimental.pallas import tpu_sc as plsc

N_NODES = 32
NC = 2
NS = 16
NW = NC * NS
L = 16


def _sc_body(ev_hbm, logits_hbm, out_hbm, ev_v, tbl_v, out_v):
    wid = lax.axis_index("s") * NC + lax.axis_index("c")
    seg = ev_hbm.shape[1] // NW
    base = wid * seg

    pltpu.sync_copy(ev_hbm.at[:, pl.ds(base, seg)], ev_v)
    pltpu.sync_copy(logits_hbm, tbl_v)

    one = jnp.ones((L,), jnp.float32)
    zero = jnp.zeros((L,), jnp.int32)
    seven = jnp.full((L,), 7, jnp.int32)

    for r in range(8):
        for o in (0, L):
            x = tbl_v[r, pl.ds(o, L)]
            tbl_v[r, pl.ds(o, L)] = one / (one + jnp.exp(-x))

    nodes = [jnp.full((L,), i, jnp.int32) for i in range(N_NODES)]

    @plsc.parallel_loop(0, seg // L, unroll=2)
    def vec_body(vb):
        off = vb * L
        e = [ev_v[i, pl.ds(off, L)] for i in range(N_NODES - 1)]
        confs = [e[0] >> 1, e[0], (e[0] << 1) | e[1]]
        confs += [(e[i - 3] << 2) | (e[i - 2] << 1) | e[i - 1]
                  for i in range(3, N_NODES)]
        for i in range(N_NODES):
            out_v[i, pl.ds(off, L)] = plsc.load_gather(tbl_v, [confs[i], nodes[i]])

    pltpu.sync_copy(out_v, out_hbm.at[:, pl.ds(base, seg)])


def kernel(evidence_tensor, logits):
    B, n = evidence_tensor.shape
    ev_t = evidence_tensor.astype(jnp.int32).T
    logits_t = logits.T
    seg = B // NW

    mesh = plsc.VectorSubcoreMesh(core_axis_name="c", subcore_axis_name="s")
    out_t = pl.kernel(
        _sc_body,
        out_type=jax.ShapeDtypeStruct((n, B), jnp.float32),
        mesh=mesh,
        compiler_params=pltpu.CompilerParams(
            needs_layout_passes=False,
            use_tc_tiling_on_sc=True,
        ),
        scratch_types=[
            pltpu.VMEM((N_NODES, seg), jnp.int32),
            pltpu.VMEM((8, N_NODES), jnp.float32),
            pltpu.VMEM((N_NODES, seg), jnp.float32),
        ],
    )(ev_t, logits_t)
    return out_t.T


# speedup vs baseline: 1.1405x; 1.1405x over previous
import jax
import jax.numpy as jnp
from jax import lax
from jax.experimental import pallas as pl
from jax.experimental.pallas import tpu as pltpu
from jax.experimental.pallas import tpu_sc as plsc

N_NODES = 32
NC = 2
NS = 16
NW = NC * NS
L = 16


def _sc_body(ev_hbm, logits_hbm, out_hbm, ev_v, tbl_v, out_v):
    wid = lax.axis_index("s") * NC + lax.axis_index("c")
    seg = ev_hbm.shape[1] // NW
    base = wid * seg

    pltpu.sync_copy(ev_hbm.at[:, pl.ds(base, seg)], ev_v)
    pltpu.sync_copy(logits_hbm, tbl_v)

    one = jnp.ones((L,), jnp.float32)
    zero = jnp.zeros((L,), jnp.int32)
    seven = jnp.full((L,), 7, jnp.int32)

    for r in range(8):
        for o in (0, L):
            x = tbl_v[r, pl.ds(o, L)]
            tbl_v[r, pl.ds(o, L)] = one / (one + jnp.exp(-x))

    nodes = [jnp.full((L,), i, jnp.int32) for i in range(N_NODES)]

    @plsc.parallel_loop(0, seg // L, unroll=1)
    def vec_body(vb):
        off = vb * L
        e = [ev_v[i, pl.ds(off, L)] for i in range(N_NODES - 1)]
        confs = [e[0] >> 1, e[0], (e[0] << 1) | e[1]]
        confs += [(e[i - 3] << 2) | (e[i - 2] << 1) | e[i - 1]
                  for i in range(3, N_NODES)]
        for i in range(N_NODES):
            out_v[i, pl.ds(off, L)] = plsc.load_gather(tbl_v, [confs[i], nodes[i]])

    pltpu.sync_copy(out_v, out_hbm.at[:, pl.ds(base, seg)])


def kernel(evidence_tensor, logits):
    B, n = evidence_tensor.shape
    ev_t = evidence_tensor.astype(jnp.int32).T
    logits_t = logits.T
    seg = B // NW

    mesh = plsc.VectorSubcoreMesh(core_axis_name="c", subcore_axis_name="s")
    out_t = pl.kernel(
        _sc_body,
        out_type=jax.ShapeDtypeStruct((n, B), jnp.float32),
        mesh=mesh,
        compiler_params=pltpu.CompilerParams(
            needs_layout_passes=False,
            use_tc_tiling_on_sc=True,
        ),
        scratch_types=[
            pltpu.VMEM((N_NODES, seg), jnp.int32),
            pltpu.VMEM((8, N_NODES), jnp.float32),
            pltpu.VMEM((N_NODES, seg), jnp.float32),
        ],
    )(ev_t, logits_t)
    return out_t.T


# speedup vs baseline: 1.1594x; 1.0166x over previous
import jax
import jax.numpy as jnp
from jax import lax
from jax.experimental import pallas as pl
from jax.experimental.pallas import tpu as pltpu
from jax.experimental.pallas import tpu_sc as plsc

N_NODES = 32
NC = 2
NS = 16
NW = NC * NS
L = 16


def _sc_body(ev_hbm, logits_hbm, out_hbm, ev_v, tbl_v, out_v):
    wid = lax.axis_index("s") * NC + lax.axis_index("c")
    seg = ev_hbm.shape[1] // NW
    base = wid * seg

    pltpu.sync_copy(ev_hbm.at[:, pl.ds(base, seg)], ev_v)
    pltpu.sync_copy(logits_hbm, tbl_v)

    one = jnp.ones((L,), jnp.float32)
    zero = jnp.zeros((L,), jnp.int32)
    seven = jnp.full((L,), 7, jnp.int32)

    for r in range(8):
        for o in (0, L):
            x = tbl_v[r, pl.ds(o, L)]
            tbl_v[r, pl.ds(o, L)] = one / (one + jnp.exp(-x))

    nodes = [jnp.full((L,), i, jnp.int32) for i in range(N_NODES)]

    @plsc.parallel_loop(0, seg // L, unroll=2)
    def vec_body(vb):
        off = vb * L
        conf = ev_v[0, pl.ds(off, L)] >> 1
        for i in range(N_NODES):
            out_v[i, pl.ds(off, L)] = plsc.load_gather(tbl_v, [conf, nodes[i]])
            if i + 1 < N_NODES:
                e = ev_v[i, pl.ds(off, L)]
                conf = ((conf << 1) | e) & seven

    pltpu.sync_copy(out_v, out_hbm.at[:, pl.ds(base, seg)])


def kernel(evidence_tensor, logits):
    B, n = evidence_tensor.shape
    ev_t = evidence_tensor.astype(jnp.int32).T
    logits_t = logits.T
    seg = B // NW

    mesh = plsc.VectorSubcoreMesh(core_axis_name="c", subcore_axis_name="s")
    out_t = pl.kernel(
        _sc_body,
        out_type=jax.ShapeDtypeStruct((n, B), jnp.float32),
        mesh=mesh,
        compiler_params=pltpu.CompilerParams(
            needs_layout_passes=False,
            use_tc_tiling_on_sc=True,
            disable_bounds_checks=True,
        ),
        scratch_types=[
            pltpu.VMEM((N_NODES, seg), jnp.int32),
            pltpu.VMEM((8, N_NODES), jnp.float32),
            pltpu.VMEM((N_NODES, seg), jnp.float32),
        ],
    )(ev_t, logits_t)
    return out_t.T
